# baseline (speedup 1.0000x reference)
import functools

import jax
import jax.numpy as jnp
from jax import lax
from jax.experimental import pallas as pl
from jax.experimental.pallas import tpu as pltpu
from jax.experimental.pallas import tpu_sc as plsc

N = 10000
E = 320000
D = 128

NC = 2
NS = 16
NW = NC * NS
EPW = E // NW
C = 40
NCHUNK = EPW // C
SUPER = 5
SUBN = NCHUNK // SUPER
NBUF = 6
RPS = 624
RTAIL = N - NS * RPS
ZROWS = 640


def _sc_segment_sum(x, src2d, dst2d, zrows, zcnt):
    mesh = plsc.VectorSubcoreMesh(
        core_axis_name="c", subcore_axis_name="s",
        num_cores=NC, num_subcores=NS,
    )

    @functools.partial(
        pl.kernel,
        out_type=(
            jax.ShapeDtypeStruct((NC, N, D), jnp.float32),
            jax.ShapeDtypeStruct((NC, 1, N), jnp.float32),
        ),
        mesh=mesh,
        scratch_types=[
            pltpu.VMEM((SUBN, C), jnp.int32),
            pltpu.VMEM((SUBN, C), jnp.int32),
            pltpu.VMEM((NBUF, C, D), jnp.float32),
            pltpu.VMEM((128,), jnp.float32),
            pltpu.VMEM_SHARED((N, D), jnp.float32),
            pltpu.VMEM_SHARED((N,), jnp.float32),
            pltpu.SemaphoreType.DMA,
            pltpu.SemaphoreType.DMA,
            pltpu.SemaphoreType.DMA,
        ],
    )
    def k(x_hbm, src_hbm, dst_hbm, zrows_hbm, zcnt_hbm,
          agg_out, cnt_out, src_v, dst_v, rows_v, ones_v,
          agg_sh, cnt_sh, gsem, ssem, osem):
        c = lax.axis_index("c")
        s = lax.axis_index("s")
        wid = c * NS + s

        pltpu.sync_copy(zrows_hbm.at[pl.ds(0, RPS)],
                        agg_sh.at[pl.ds(s * RPS, RPS)])

        @pl.when(s == NS - 1)
        def _():
            pltpu.sync_copy(zrows_hbm.at[pl.ds(0, RTAIL)],
                            agg_sh.at[pl.ds(NS * RPS, RTAIL)])

        @pl.when(s == 0)
        def _():
            pltpu.sync_copy(zcnt_hbm, cnt_sh)

        for t in range(8):
            ones_v[pl.ds(t * 16, 16)] = jnp.ones((16,), jnp.float32)
        ones_c = ones_v.at[pl.ds(0, C)]

        plsc.subcore_barrier()

        for g in range(SUPER):
            pltpu.sync_copy(src_hbm.at[wid].at[g], src_v)
            pltpu.sync_copy(dst_hbm.at[wid].at[g], dst_v)
            for p in range(NBUF - 1):
                pltpu.async_copy(x_hbm.at[src_v.at[p]], rows_v.at[p], gsem)

            def body(j, carry):
                buf = lax.rem(j, NBUF)
                pltpu.make_async_copy(x_hbm.at[src_v.at[j]],
                                      rows_v.at[buf], gsem).wait()

                pltpu.async_copy(rows_v.at[buf], agg_sh.at[dst_v.at[j]],
                                 ssem, add=True)
                pltpu.async_copy(ones_c, cnt_sh.at[dst_v.at[j]],
                                 osem, add=True)

                nbuf = lax.rem(j + NBUF - 1, NBUF)

                @pl.when(j > 0)
                def _():
                    pltpu.make_async_copy(rows_v.at[nbuf],
                                          agg_sh.at[dst_v.at[j - 1]],
                                          ssem).wait()

                @pl.when(j < SUBN - (NBUF - 1))
                def _():
                    pltpu.async_copy(x_hbm.at[src_v.at[j + NBUF - 1]],
                                     rows_v.at[nbuf], gsem)

                return carry

            lax.fori_loop(0, SUBN, body, 0, unroll=False)

            pltpu.make_async_copy(rows_v.at[0], agg_sh.at[dst_v.at[0]],
                                  ssem).wait()

            def drain(j, carry):
                pltpu.make_async_copy(ones_c, cnt_sh.at[dst_v.at[0]],
                                      osem).wait()
                return carry

            lax.fori_loop(0, SUBN, drain, 0, unroll=False)

        plsc.subcore_barrier()

        pltpu.sync_copy(agg_sh.at[pl.ds(s * RPS, RPS)],
                        agg_out.at[c].at[pl.ds(s * RPS, RPS)])

        @pl.when(s == NS - 1)
        def _():
            pltpu.sync_copy(agg_sh.at[pl.ds(NS * RPS, RTAIL)],
                            agg_out.at[c].at[pl.ds(NS * RPS, RTAIL)])

        @pl.when(s == 0)
        def _():
            pltpu.sync_copy(cnt_sh, cnt_out.at[c].at[0])

    return k(x, src2d, dst2d, zrows, zcnt)


NB = 1000
GRID = N // NB


def _tc_body(p_ref, cnt_ref, x_ref, wl_ref, wr_ref, bl_ref, w2_ref, b2_ref,
             o_ref, acc_ref):
    i = pl.program_id(0)
    ssum = p_ref[0] + p_ref[1]
    cnt = cnt_ref[0, 0, 0] + cnt_ref[1, 0, 0]
    cnt = jnp.maximum(cnt, 1.0)
    agg = ssum / cnt[:, None]
    dn = (((1,), (1,)), ((), ()))
    h = (lax.dot_general(agg, wl_ref[...], dn,
                         preferred_element_type=jnp.float32)
         + lax.dot_general(x_ref[...], wr_ref[...], dn,
                           preferred_element_type=jnp.float32)
         + bl_ref[...])
    h = jnp.maximum(h, 0.0)
    hs = jnp.sum(h, axis=0, keepdims=True)

    @pl.when(i == 0)
    def _():
        acc_ref[...] = hs

    @pl.when(i > 0)
    def _():
        acc_ref[...] = acc_ref[...] + hs

    @pl.when(i == GRID - 1)
    def _():
        hmean = acc_ref[...] * (1.0 / N)
        o_ref[...] = (lax.dot_general(hmean, w2_ref[...], dn,
                                      preferred_element_type=jnp.float32)
                      + b2_ref[...])


def kernel(x, edge_index, W_l, b_l, W_r, W2, b2):
    src2d = edge_index[0].reshape(NW, SUPER, SUBN, C)
    dst2d = edge_index[1].reshape(NW, SUPER, SUBN, C)
    zrows = jnp.zeros((ZROWS, D), jnp.float32)
    zcnt = jnp.zeros((N,), jnp.float32)

    agg_p, cnt_p = _sc_segment_sum(x, src2d, dst2d, zrows, zcnt)

    y = pl.pallas_call(
        _tc_body,
        grid=(GRID,),
        in_specs=[
            pl.BlockSpec((NC, NB, D), lambda i: (0, i, 0)),
            pl.BlockSpec((NC, 1, 1, NB), lambda i: (0, i, 0, 0)),
            pl.BlockSpec((NB, D), lambda i: (i, 0)),
            pl.BlockSpec((D, D), lambda i: (0, 0)),
            pl.BlockSpec((D, D), lambda i: (0, 0)),
            pl.BlockSpec((1, D), lambda i: (0, 0)),
            pl.BlockSpec((D, D), lambda i: (0, 0)),
            pl.BlockSpec((1, D), lambda i: (0, 0)),
        ],
        out_specs=pl.BlockSpec((1, D), lambda i: (0, 0)),
        out_shape=jax.ShapeDtypeStruct((1, D), jnp.float32),
        scratch_shapes=[pltpu.VMEM((1, D), jnp.float32)],
    )(agg_p, cnt_p.reshape(NC, GRID, 1, NB), x, W_l, W_r,
      b_l.reshape(1, D), W2, b2.reshape(1, D))

    return y[0]


# speedup vs baseline: 1.0175x; 1.0175x over previous
import functools

import jax
import jax.numpy as jnp
from jax import lax
from jax.experimental import pallas as pl
from jax.experimental.pallas import tpu as pltpu
from jax.experimental.pallas import tpu_sc as plsc

N = 10000
E = 320000
D = 128

NC = 2
NS = 16
NW = NC * NS
EPW = E // NW
C = 80
NCHUNK = EPW // C
SUPER = 5
SUBN = NCHUNK // SUPER
NBUF = 3
RPS = 624
RTAIL = N - NS * RPS
ZROWS = 640


def _sc_segment_sum(x, src2d, dst2d, zrows, zcnt):
    mesh = plsc.VectorSubcoreMesh(
        core_axis_name="c", subcore_axis_name="s",
        num_cores=NC, num_subcores=NS,
    )

    @functools.partial(
        pl.kernel,
        out_type=(
            jax.ShapeDtypeStruct((NC, N, D), jnp.float32),
            jax.ShapeDtypeStruct((NC, 1, N), jnp.float32),
        ),
        mesh=mesh,
        scratch_types=[
            pltpu.VMEM((2, SUBN, C), jnp.int32),
            pltpu.VMEM((2, SUBN, C), jnp.int32),
            pltpu.VMEM((NBUF, C, D), jnp.float32),
            pltpu.VMEM((128,), jnp.float32),
            pltpu.VMEM_SHARED((N, D), jnp.float32),
            pltpu.VMEM_SHARED((N,), jnp.float32),
            pltpu.SemaphoreType.DMA,
            pltpu.SemaphoreType.DMA,
            pltpu.SemaphoreType.DMA,
            pltpu.SemaphoreType.DMA,
        ],
    )
    def k(x_hbm, src_hbm, dst_hbm, zrows_hbm, zcnt_hbm,
          agg_out, cnt_out, src_v2, dst_v2, rows_v, ones_v,
          agg_sh, cnt_sh, gsem, ssem, osem, isem):
        c = lax.axis_index("c")
        s = lax.axis_index("s")
        wid = c * NS + s

        pltpu.sync_copy(zrows_hbm.at[pl.ds(0, RPS)],
                        agg_sh.at[pl.ds(s * RPS, RPS)])

        @pl.when(s == NS - 1)
        def _():
            pltpu.sync_copy(zrows_hbm.at[pl.ds(0, RTAIL)],
                            agg_sh.at[pl.ds(NS * RPS, RTAIL)])

        @pl.when(s == 0)
        def _():
            pltpu.sync_copy(zcnt_hbm, cnt_sh)

        for t in range(8):
            ones_v[pl.ds(t * 16, 16)] = jnp.ones((16,), jnp.float32)
        ones_c = ones_v.at[pl.ds(0, C)]

        pltpu.async_copy(src_hbm.at[wid].at[0], src_v2.at[0], isem)
        pltpu.async_copy(dst_hbm.at[wid].at[0], dst_v2.at[0], isem)

        plsc.subcore_barrier()

        for g in range(SUPER):
            gb = g % 2
            src_v = src_v2.at[gb]
            dst_v = dst_v2.at[gb]
            pltpu.make_async_copy(src_hbm.at[wid].at[g], src_v, isem).wait()
            pltpu.make_async_copy(dst_hbm.at[wid].at[g], dst_v, isem).wait()
            if g + 1 < SUPER:
                pltpu.async_copy(src_hbm.at[wid].at[g + 1],
                                 src_v2.at[1 - gb], isem)
                pltpu.async_copy(dst_hbm.at[wid].at[g + 1],
                                 dst_v2.at[1 - gb], isem)
            for p in range(NBUF - 1):
                pltpu.async_copy(x_hbm.at[src_v.at[p]], rows_v.at[p], gsem)

            def body(j, carry):
                buf = lax.rem(j, NBUF)
                pltpu.make_async_copy(x_hbm.at[src_v.at[j]],
                                      rows_v.at[buf], gsem).wait()

                pltpu.async_copy(rows_v.at[buf], agg_sh.at[dst_v.at[j]],
                                 ssem, add=True)
                pltpu.async_copy(ones_c, cnt_sh.at[dst_v.at[j]],
                                 osem, add=True)

                nbuf = lax.rem(j + NBUF - 1, NBUF)

                @pl.when(j > 0)
                def _():
                    pltpu.make_async_copy(rows_v.at[nbuf],
                                          agg_sh.at[dst_v.at[j - 1]],
                                          ssem).wait()

                @pl.when(j < SUBN - (NBUF - 1))
                def _():
                    pltpu.async_copy(x_hbm.at[src_v.at[j + NBUF - 1]],
                                     rows_v.at[nbuf], gsem)

                return carry

            lax.fori_loop(0, SUBN, body, 0, unroll=False)

            pltpu.make_async_copy(rows_v.at[0], agg_sh.at[dst_v.at[0]],
                                  ssem).wait()

            def drain(j, carry):
                pltpu.make_async_copy(ones_c, cnt_sh.at[dst_v.at[0]],
                                      osem).wait()
                return carry

            lax.fori_loop(0, SUBN, drain, 0, unroll=False)

        plsc.subcore_barrier()

        pltpu.sync_copy(agg_sh.at[pl.ds(s * RPS, RPS)],
                        agg_out.at[c].at[pl.ds(s * RPS, RPS)])

        @pl.when(s == NS - 1)
        def _():
            pltpu.sync_copy(agg_sh.at[pl.ds(NS * RPS, RTAIL)],
                            agg_out.at[c].at[pl.ds(NS * RPS, RTAIL)])

        @pl.when(s == 0)
        def _():
            pltpu.sync_copy(cnt_sh, cnt_out.at[c].at[0])

    return k(x, src2d, dst2d, zrows, zcnt)


NB = 1000
GRID = N // NB


def _tc_xr_body(x_ref, wr_ref, bl_ref, o_ref):
    dn = (((1,), (1,)), ((), ()))
    o_ref[...] = (lax.dot_general(x_ref[...], wr_ref[...], dn,
                                  preferred_element_type=jnp.float32)
                  + bl_ref[...])


def _tc_body(p_ref, cnt_ref, xr_ref, wl_ref, w2_ref, b2_ref,
             o_ref, acc_ref):
    i = pl.program_id(0)
    ssum = p_ref[0] + p_ref[1]
    cnt = cnt_ref[0, 0, 0] + cnt_ref[1, 0, 0]
    cnt = jnp.maximum(cnt, 1.0)
    agg = ssum / cnt[:, None]
    dn = (((1,), (1,)), ((), ()))
    h = (lax.dot_general(agg, wl_ref[...], dn,
                         preferred_element_type=jnp.float32)
         + xr_ref[...])
    h = jnp.maximum(h, 0.0)
    hs = jnp.sum(h, axis=0, keepdims=True)

    @pl.when(i == 0)
    def _():
        acc_ref[...] = hs

    @pl.when(i > 0)
    def _():
        acc_ref[...] = acc_ref[...] + hs

    @pl.when(i == GRID - 1)
    def _():
        hmean = acc_ref[...] * (1.0 / N)
        o_ref[...] = (lax.dot_general(hmean, w2_ref[...], dn,
                                      preferred_element_type=jnp.float32)
                      + b2_ref[...])


def kernel(x, edge_index, W_l, b_l, W_r, W2, b2):
    src2d = edge_index[0].reshape(NW, SUPER, SUBN, C)
    dst2d = edge_index[1].reshape(NW, SUPER, SUBN, C)
    zrows = jnp.zeros((ZROWS, D), jnp.float32)
    zcnt = jnp.zeros((N,), jnp.float32)

    agg_p, cnt_p = _sc_segment_sum(x, src2d, dst2d, zrows, zcnt)

    xr = pl.pallas_call(
        _tc_xr_body,
        grid=(GRID,),
        in_specs=[
            pl.BlockSpec((NB, D), lambda i: (i, 0)),
            pl.BlockSpec((D, D), lambda i: (0, 0)),
            pl.BlockSpec((1, D), lambda i: (0, 0)),
        ],
        out_specs=pl.BlockSpec((NB, D), lambda i: (i, 0)),
        out_shape=jax.ShapeDtypeStruct((N, D), jnp.float32),
    )(x, W_r, b_l.reshape(1, D))

    y = pl.pallas_call(
        _tc_body,
        grid=(GRID,),
        in_specs=[
            pl.BlockSpec((NC, NB, D), lambda i: (0, i, 0)),
            pl.BlockSpec((NC, 1, 1, NB), lambda i: (0, i, 0, 0)),
            pl.BlockSpec((NB, D), lambda i: (i, 0)),
            pl.BlockSpec((D, D), lambda i: (0, 0)),
            pl.BlockSpec((D, D), lambda i: (0, 0)),
            pl.BlockSpec((1, D), lambda i: (0, 0)),
        ],
        out_specs=pl.BlockSpec((1, D), lambda i: (0, 0)),
        out_shape=jax.ShapeDtypeStruct((1, D), jnp.float32),
        scratch_shapes=[pltpu.VMEM((1, D), jnp.float32)],
    )(agg_p, cnt_p.reshape(NC, GRID, 1, NB), xr, W_l, W2, b2.reshape(1, D))

    return y[0]


# speedup vs baseline: 1.0330x; 1.0152x over previous
import functools

import jax
import jax.numpy as jnp
from jax import lax
from jax.experimental import pallas as pl
from jax.experimental.pallas import tpu as pltpu
from jax.experimental.pallas import tpu_sc as plsc

N = 10000
E = 320000
D = 128

NC = 2
NS = 16
NW = NC * NS
EPW = E // NW
C = 80
NCHUNK = EPW // C
SUPER = 5
SUBN = NCHUNK // SUPER
NBUF = 3
RPS = 624
RTAIL = N - NS * RPS
ZROWS = 640


def _sc_segment_sum(x, src2d, dst2d, zrows, zcnt):
    mesh = plsc.VectorSubcoreMesh(
        core_axis_name="c", subcore_axis_name="s",
        num_cores=NC, num_subcores=NS,
    )

    @functools.partial(
        pl.kernel,
        out_type=(
            jax.ShapeDtypeStruct((NC, N, D), jnp.float32),
            jax.ShapeDtypeStruct((NC, 1, N), jnp.float32),
        ),
        mesh=mesh,
        scratch_types=[
            pltpu.VMEM((2, SUBN, C), jnp.int32),
            pltpu.VMEM((2, SUBN, C), jnp.int32),
            pltpu.VMEM((NBUF, C, D), jnp.float32),
            pltpu.VMEM((128,), jnp.float32),
            pltpu.VMEM_SHARED((N, D), jnp.float32),
            pltpu.VMEM_SHARED((N,), jnp.float32),
            pltpu.SemaphoreType.DMA,
            pltpu.SemaphoreType.DMA,
            pltpu.SemaphoreType.DMA,
            pltpu.SemaphoreType.DMA,
        ],
    )
    def k(x_hbm, src_hbm, dst_hbm, zrows_hbm, zcnt_hbm,
          agg_out, cnt_out, src_v2, dst_v2, rows_v, ones_v,
          agg_sh, cnt_sh, gsem, ssem, osem, isem):
        c = lax.axis_index("c")
        s = lax.axis_index("s")
        wid = c * NS + s

        pltpu.sync_copy(zrows_hbm.at[pl.ds(0, RPS)],
                        agg_sh.at[pl.ds(s * RPS, RPS)])

        @pl.when(s == NS - 1)
        def _():
            pltpu.sync_copy(zrows_hbm.at[pl.ds(0, RTAIL)],
                            agg_sh.at[pl.ds(NS * RPS, RTAIL)])

        @pl.when(s == 0)
        def _():
            pltpu.sync_copy(zcnt_hbm, cnt_sh)

        for t in range(8):
            ones_v[pl.ds(t * 16, 16)] = jnp.ones((16,), jnp.float32)
        ones_c = ones_v.at[pl.ds(0, C)]

        pltpu.async_copy(src_hbm.at[wid].at[0], src_v2.at[0], isem)
        pltpu.async_copy(dst_hbm.at[wid].at[0], dst_v2.at[0], isem)

        plsc.subcore_barrier()

        for g in range(SUPER):
            gb = g % 2
            src_v = src_v2.at[gb]
            dst_v = dst_v2.at[gb]
            pltpu.make_async_copy(src_hbm.at[wid].at[g], src_v, isem).wait()
            pltpu.make_async_copy(dst_hbm.at[wid].at[g], dst_v, isem).wait()
            if g + 1 < SUPER:
                pltpu.async_copy(src_hbm.at[wid].at[g + 1],
                                 src_v2.at[1 - gb], isem)
                pltpu.async_copy(dst_hbm.at[wid].at[g + 1],
                                 dst_v2.at[1 - gb], isem)
            for p in range(NBUF - 1):
                pltpu.async_copy(x_hbm.at[src_v.at[p]], rows_v.at[p], gsem)

            def body(j, carry):
                buf = lax.rem(j, NBUF)
                pltpu.make_async_copy(x_hbm.at[src_v.at[j]],
                                      rows_v.at[buf], gsem).wait()

                pltpu.async_copy(rows_v.at[buf], agg_sh.at[dst_v.at[j]],
                                 ssem, add=True)
                pltpu.async_copy(ones_c, cnt_sh.at[dst_v.at[j]],
                                 osem, add=True)

                nbuf = lax.rem(j + NBUF - 1, NBUF)

                @pl.when(j > 0)
                def _():
                    pltpu.make_async_copy(rows_v.at[nbuf],
                                          agg_sh.at[dst_v.at[j - 1]],
                                          ssem).wait()

                @pl.when(j < SUBN - (NBUF - 1))
                def _():
                    pltpu.async_copy(x_hbm.at[src_v.at[j + NBUF - 1]],
                                     rows_v.at[nbuf], gsem)

                return carry

            lax.fori_loop(0, SUBN, body, 0, unroll=False)

            pltpu.make_async_copy(rows_v.at[0], agg_sh.at[dst_v.at[0]],
                                  ssem).wait()

            def drain(j, carry):
                pltpu.make_async_copy(ones_c, cnt_sh.at[dst_v.at[0]],
                                      osem).wait()
                return carry

            lax.fori_loop(0, SUBN, drain, 0, unroll=False)

        plsc.subcore_barrier()

        pltpu.sync_copy(agg_sh.at[pl.ds(s * RPS, RPS)],
                        agg_out.at[c].at[pl.ds(s * RPS, RPS)])

        @pl.when(s == NS - 1)
        def _():
            pltpu.sync_copy(agg_sh.at[pl.ds(NS * RPS, RTAIL)],
                            agg_out.at[c].at[pl.ds(NS * RPS, RTAIL)])

        @pl.when(s == 0)
        def _():
            pltpu.sync_copy(cnt_sh, cnt_out.at[c].at[0])

    return k(x, src2d, dst2d, zrows, zcnt)


NB = 2000
GRID = N // NB


def _tc_xr_body(x_ref, wr_ref, bl_ref, o_ref):
    dn = (((1,), (1,)), ((), ()))
    o_ref[...] = (lax.dot_general(x_ref[...], wr_ref[...], dn,
                                  preferred_element_type=jnp.float32)
                  + bl_ref[...])


def _tc_body(p_ref, cnt_ref, xr_ref, wl_ref, w2_ref, b2_ref,
             o_ref, acc_ref):
    i = pl.program_id(0)
    ssum = p_ref[0] + p_ref[1]
    cnt = cnt_ref[0, 0, 0] + cnt_ref[1, 0, 0]
    cnt = jnp.maximum(cnt, 1.0)
    agg = ssum / cnt[:, None]
    dn = (((1,), (1,)), ((), ()))
    h = (lax.dot_general(agg, wl_ref[...], dn,
                         preferred_element_type=jnp.float32)
         + xr_ref[...])
    h = jnp.maximum(h, 0.0)
    hs = jnp.sum(h, axis=0, keepdims=True)

    @pl.when(i == 0)
    def _():
        acc_ref[...] = hs

    @pl.when(i > 0)
    def _():
        acc_ref[...] = acc_ref[...] + hs

    @pl.when(i == GRID - 1)
    def _():
        hmean = acc_ref[...] * (1.0 / N)
        o_ref[...] = (lax.dot_general(hmean, w2_ref[...], dn,
                                      preferred_element_type=jnp.float32)
                      + b2_ref[...])


def kernel(x, edge_index, W_l, b_l, W_r, W2, b2):
    src2d = edge_index[0].reshape(NW, SUPER, SUBN, C)
    dst2d = edge_index[1].reshape(NW, SUPER, SUBN, C)
    zrows = jnp.zeros((ZROWS, D), jnp.float32)
    zcnt = jnp.zeros((N,), jnp.float32)

    agg_p, cnt_p = _sc_segment_sum(x, src2d, dst2d, zrows, zcnt)

    xr = pl.pallas_call(
        _tc_xr_body,
        grid=(GRID,),
        in_specs=[
            pl.BlockSpec((NB, D), lambda i: (i, 0)),
            pl.BlockSpec((D, D), lambda i: (0, 0)),
            pl.BlockSpec((1, D), lambda i: (0, 0)),
        ],
        out_specs=pl.BlockSpec((NB, D), lambda i: (i, 0)),
        out_shape=jax.ShapeDtypeStruct((N, D), jnp.float32),
    )(x, W_r, b_l.reshape(1, D))

    y = pl.pallas_call(
        _tc_body,
        grid=(GRID,),
        in_specs=[
            pl.BlockSpec((NC, NB, D), lambda i: (0, i, 0)),
            pl.BlockSpec((NC, 1, 1, NB), lambda i: (0, i, 0, 0)),
            pl.BlockSpec((NB, D), lambda i: (i, 0)),
            pl.BlockSpec((D, D), lambda i: (0, 0)),
            pl.BlockSpec((D, D), lambda i: (0, 0)),
            pl.BlockSpec((1, D), lambda i: (0, 0)),
        ],
        out_specs=pl.BlockSpec((1, D), lambda i: (0, 0)),
        out_shape=jax.ShapeDtypeStruct((1, D), jnp.float32),
        scratch_shapes=[pltpu.VMEM((1, D), jnp.float32)],
    )(agg_p, cnt_p.reshape(NC, GRID, 1, NB), xr, W_l, W2, b2.reshape(1, D))

    return y[0]


# speedup vs baseline: 1.0567x; 1.0230x over previous
import functools

import jax
import jax.numpy as jnp
from jax import lax
from jax.experimental import pallas as pl
from jax.experimental.pallas import tpu as pltpu
from jax.experimental.pallas import tpu_sc as plsc

N = 10000
E = 320000
D = 128

NC = 2
NS = 16
NW = NC * NS
EPW = E // NW
C = 80
NCHUNK = EPW // C
SUPER = 5
SUBN = NCHUNK // SUPER
NBUF = 3
RPS = 624
RTAIL = N - NS * RPS
ZROWS = 640


def _sc_segment_sum(x, src2d, dst2d, zrows, zcnt):
    mesh = plsc.VectorSubcoreMesh(
        core_axis_name="c", subcore_axis_name="s",
        num_cores=NC, num_subcores=NS,
    )

    @functools.partial(
        pl.kernel,
        out_type=(
            jax.ShapeDtypeStruct((NC, N, D), jnp.float32),
            jax.ShapeDtypeStruct((NC, 1, N), jnp.float32),
        ),
        mesh=mesh,
        scratch_types=[
            pltpu.VMEM((2, SUBN, C), jnp.int32),
            pltpu.VMEM((2, SUBN, C), jnp.int32),
            pltpu.VMEM((NBUF, C, D), jnp.float32),
            pltpu.VMEM((128,), jnp.float32),
            pltpu.VMEM_SHARED((N, D), jnp.float32),
            pltpu.VMEM_SHARED((N,), jnp.float32),
            pltpu.SemaphoreType.DMA,
            pltpu.SemaphoreType.DMA,
            pltpu.SemaphoreType.DMA,
            pltpu.SemaphoreType.DMA,
        ],
    )
    def k(x_hbm, src_hbm, dst_hbm, zrows_hbm, zcnt_hbm,
          agg_out, cnt_out, src_v2, dst_v2, rows_v, ones_v,
          agg_sh, cnt_sh, gsem, ssem, osem, isem):
        c = lax.axis_index("c")
        s = lax.axis_index("s")
        wid = c * NS + s

        pltpu.sync_copy(zrows_hbm.at[pl.ds(0, RPS)],
                        agg_sh.at[pl.ds(s * RPS, RPS)])

        @pl.when(s == NS - 1)
        def _():
            pltpu.sync_copy(zrows_hbm.at[pl.ds(0, RTAIL)],
                            agg_sh.at[pl.ds(NS * RPS, RTAIL)])

        @pl.when(s == 0)
        def _():
            pltpu.sync_copy(zcnt_hbm, cnt_sh)

        for t in range(8):
            ones_v[pl.ds(t * 16, 16)] = jnp.ones((16,), jnp.float32)
        ones_c = ones_v.at[pl.ds(0, C)]

        pltpu.async_copy(src_hbm.at[wid].at[0], src_v2.at[0], isem)
        pltpu.async_copy(dst_hbm.at[wid].at[0], dst_v2.at[0], isem)

        plsc.subcore_barrier()

        for g in range(SUPER):
            gb = g % 2
            src_v = src_v2.at[gb]
            dst_v = dst_v2.at[gb]
            pltpu.make_async_copy(src_hbm.at[wid].at[g], src_v, isem).wait()
            pltpu.make_async_copy(dst_hbm.at[wid].at[g], dst_v, isem).wait()
            if g + 1 < SUPER:
                pltpu.async_copy(src_hbm.at[wid].at[g + 1],
                                 src_v2.at[1 - gb], isem)
                pltpu.async_copy(dst_hbm.at[wid].at[g + 1],
                                 dst_v2.at[1 - gb], isem)
            for p in range(NBUF - 1):
                pltpu.async_copy(x_hbm.at[src_v.at[p]], rows_v.at[p], gsem)

            def body(j, carry):
                buf = lax.rem(j, NBUF)
                pltpu.make_async_copy(x_hbm.at[src_v.at[j]],
                                      rows_v.at[buf], gsem).wait()

                pltpu.async_copy(rows_v.at[buf], agg_sh.at[dst_v.at[j]],
                                 ssem, add=True)
                pltpu.async_copy(ones_c, cnt_sh.at[dst_v.at[j]],
                                 osem, add=True)

                nbuf = lax.rem(j + NBUF - 1, NBUF)

                @pl.when(j > 0)
                def _():
                    pltpu.make_async_copy(rows_v.at[nbuf],
                                          agg_sh.at[dst_v.at[j - 1]],
                                          ssem).wait()

                @pl.when(j < SUBN - (NBUF - 1))
                def _():
                    pltpu.async_copy(x_hbm.at[src_v.at[j + NBUF - 1]],
                                     rows_v.at[nbuf], gsem)

                return carry

            lax.fori_loop(0, SUBN, body, 0, unroll=False)

            pltpu.make_async_copy(rows_v.at[0], agg_sh.at[dst_v.at[0]],
                                  ssem).wait()

            def drain(j, carry):
                pltpu.make_async_copy(ones_c, cnt_sh.at[dst_v.at[0]],
                                      osem).wait()
                return carry

            lax.fori_loop(0, SUBN, drain, 0, unroll=False)

        plsc.subcore_barrier()

        pltpu.sync_copy(agg_sh.at[pl.ds(s * RPS, RPS)],
                        agg_out.at[c].at[pl.ds(s * RPS, RPS)])

        @pl.when(s == NS - 1)
        def _():
            pltpu.sync_copy(agg_sh.at[pl.ds(NS * RPS, RTAIL)],
                            agg_out.at[c].at[pl.ds(NS * RPS, RTAIL)])

        @pl.when(s == 0)
        def _():
            pltpu.sync_copy(cnt_sh, cnt_out.at[c].at[0])

    return k(x, src2d, dst2d, zrows, zcnt)


NB = 2000
GRID = N // NB


def _tc_body(p_ref, cnt_ref, xr_ref, wl_ref, wr_ref, w2_ref, bl_ref, b2_ref,
             o_ref, acc_ref):
    i = pl.program_id(0)
    ssum = p_ref[0] + p_ref[1]
    cnt = cnt_ref[0, 0, 0] + cnt_ref[1, 0, 0]
    cnt = jnp.maximum(cnt, 1.0)
    agg = ssum / cnt[:, None]
    dn = (((1,), (1,)), ((), ()))
    h = (lax.dot_general(agg, wl_ref[...], dn,
                         preferred_element_type=jnp.float32)
         + lax.dot_general(xr_ref[...], wr_ref[...], dn,
                           preferred_element_type=jnp.float32)
         + bl_ref[...])
    h = jnp.maximum(h, 0.0)
    hs = jnp.sum(h, axis=0, keepdims=True)

    @pl.when(i == 0)
    def _():
        acc_ref[...] = hs

    @pl.when(i > 0)
    def _():
        acc_ref[...] = acc_ref[...] + hs

    @pl.when(i == GRID - 1)
    def _():
        hmean = acc_ref[...] * (1.0 / N)
        o_ref[...] = (lax.dot_general(hmean, w2_ref[...], dn,
                                      preferred_element_type=jnp.float32)
                      + b2_ref[...])


def kernel(x, edge_index, W_l, b_l, W_r, W2, b2):
    src2d = edge_index[0].reshape(NW, SUPER, SUBN, C)
    dst2d = edge_index[1].reshape(NW, SUPER, SUBN, C)
    zrows = jnp.zeros((ZROWS, D), jnp.float32)
    zcnt = jnp.zeros((N,), jnp.float32)

    agg_p, cnt_p = _sc_segment_sum(x, src2d, dst2d, zrows, zcnt)

    y = pl.pallas_call(
        _tc_body,
        grid=(GRID,),
        in_specs=[
            pl.BlockSpec((NC, NB, D), lambda i: (0, i, 0)),
            pl.BlockSpec((NC, 1, 1, NB), lambda i: (0, i, 0, 0)),
            pl.BlockSpec((NB, D), lambda i: (i, 0)),
            pl.BlockSpec((D, D), lambda i: (0, 0)),
            pl.BlockSpec((D, D), lambda i: (0, 0)),
            pl.BlockSpec((D, D), lambda i: (0, 0)),
            pl.BlockSpec((1, D), lambda i: (0, 0)),
            pl.BlockSpec((1, D), lambda i: (0, 0)),
        ],
        out_specs=pl.BlockSpec((1, D), lambda i: (0, 0)),
        out_shape=jax.ShapeDtypeStruct((1, D), jnp.float32),
        scratch_shapes=[pltpu.VMEM((1, D), jnp.float32)],
    )(agg_p, cnt_p.reshape(NC, GRID, 1, NB), x, W_l, W_r, W2,
      b_l.reshape(1, D), b2.reshape(1, D))

    return y[0]


# speedup vs baseline: 1.0586x; 1.0017x over previous
import functools

import jax
import jax.numpy as jnp
from jax import lax
from jax.experimental import pallas as pl
from jax.experimental.pallas import tpu as pltpu
from jax.experimental.pallas import tpu_sc as plsc

N = 10000
E = 320000
D = 128

NC = 2
NS = 16
NW = NC * NS
EPW = E // NW
C = 80
NCHUNK = EPW // C
SUPER = 5
SUBN = NCHUNK // SUPER
NBUF = 3
RPS = 624
RTAIL = N - NS * RPS
ZROWS = 640


def _sc_segment_sum(x, src2d, dst2d, zrows, zcnt):
    mesh = plsc.VectorSubcoreMesh(
        core_axis_name="c", subcore_axis_name="s",
        num_cores=NC, num_subcores=NS,
    )

    @functools.partial(
        pl.kernel,
        out_type=(
            jax.ShapeDtypeStruct((NC, N, D), jnp.float32),
            jax.ShapeDtypeStruct((NC, 1, N), jnp.float32),
        ),
        mesh=mesh,
        scratch_types=[
            pltpu.VMEM((2, SUBN, C), jnp.int32),
            pltpu.VMEM((2, SUBN, C), jnp.int32),
            pltpu.VMEM((NBUF, C, D), jnp.float32),
            pltpu.VMEM((128,), jnp.float32),
            pltpu.VMEM_SHARED((N, D), jnp.float32),
            pltpu.VMEM_SHARED((N,), jnp.float32),
            pltpu.SemaphoreType.DMA,
            pltpu.SemaphoreType.DMA,
            pltpu.SemaphoreType.DMA,
            pltpu.SemaphoreType.DMA,
        ],
    )
    def k(x_hbm, src_hbm, dst_hbm, zrows_hbm, zcnt_hbm,
          agg_out, cnt_out, src_v2, dst_v2, rows_v, ones_v,
          agg_sh, cnt_sh, gsem, ssem, osem, isem):
        c = lax.axis_index("c")
        s = lax.axis_index("s")
        wid = c * NS + s

        pltpu.sync_copy(zrows_hbm.at[pl.ds(0, RPS)],
                        agg_sh.at[pl.ds(s * RPS, RPS)])

        @pl.when(s == NS - 1)
        def _():
            pltpu.sync_copy(zrows_hbm.at[pl.ds(0, RTAIL)],
                            agg_sh.at[pl.ds(NS * RPS, RTAIL)])

        @pl.when(s == 0)
        def _():
            pltpu.sync_copy(zcnt_hbm, cnt_sh)

        for t in range(8):
            ones_v[pl.ds(t * 16, 16)] = jnp.ones((16,), jnp.float32)
        ones_c = ones_v.at[pl.ds(0, C)]

        pltpu.async_copy(src_hbm.at[wid].at[0], src_v2.at[0], isem)
        pltpu.async_copy(dst_hbm.at[wid].at[0], dst_v2.at[0], isem)

        plsc.subcore_barrier()

        pltpu.make_async_copy(src_hbm.at[wid].at[0], src_v2.at[0],
                              isem).wait()
        pltpu.make_async_copy(dst_hbm.at[wid].at[0], dst_v2.at[0],
                              isem).wait()
        for p in range(NBUF - 1):
            pltpu.async_copy(x_hbm.at[src_v2.at[0].at[p]], rows_v.at[p],
                             gsem)
        if SUPER > 1:
            pltpu.async_copy(src_hbm.at[wid].at[1], src_v2.at[1], isem)
            pltpu.async_copy(dst_hbm.at[wid].at[1], dst_v2.at[1], isem)

        for g in range(SUPER):
            gb = g % 2
            src_v = src_v2.at[gb]
            dst_v = dst_v2.at[gb]

            def body(j, carry):
                buf = lax.rem(j, NBUF)
                pltpu.make_async_copy(x_hbm.at[src_v.at[j]],
                                      rows_v.at[buf], gsem).wait()

                pltpu.async_copy(rows_v.at[buf], agg_sh.at[dst_v.at[j]],
                                 ssem, add=True)
                pltpu.async_copy(ones_c, cnt_sh.at[dst_v.at[j]],
                                 osem, add=True)

                nbuf = lax.rem(j + NBUF - 1, NBUF)

                @pl.when(j > 0)
                def _():
                    pltpu.make_async_copy(rows_v.at[nbuf],
                                          agg_sh.at[dst_v.at[j - 1]],
                                          ssem).wait()

                @pl.when(j < SUBN - (NBUF - 1))
                def _():
                    pltpu.async_copy(x_hbm.at[src_v.at[j + NBUF - 1]],
                                     rows_v.at[nbuf], gsem)

                return carry

            lax.fori_loop(0, SUBN, body, 0, unroll=False)

            pltpu.make_async_copy(rows_v.at[(SUBN - 1) % NBUF],
                                  agg_sh.at[dst_v.at[0]], ssem).wait()

            if g + 1 < SUPER:
                nsrc = src_v2.at[1 - gb]
                pltpu.make_async_copy(src_hbm.at[wid].at[g + 1], nsrc,
                                      isem).wait()
                pltpu.make_async_copy(dst_hbm.at[wid].at[g + 1],
                                      dst_v2.at[1 - gb], isem).wait()
                for p in range(NBUF - 1):
                    pltpu.async_copy(x_hbm.at[nsrc.at[p]], rows_v.at[p],
                                     gsem)

            def drain(j, carry):
                pltpu.make_async_copy(ones_c, cnt_sh.at[dst_v.at[0]],
                                      osem).wait()
                return carry

            lax.fori_loop(0, SUBN, drain, 0, unroll=False)

            if g + 2 < SUPER:
                pltpu.async_copy(src_hbm.at[wid].at[g + 2], src_v2.at[gb],
                                 isem)
                pltpu.async_copy(dst_hbm.at[wid].at[g + 2], dst_v2.at[gb],
                                 isem)

        plsc.subcore_barrier()

        pltpu.sync_copy(agg_sh.at[pl.ds(s * RPS, RPS)],
                        agg_out.at[c].at[pl.ds(s * RPS, RPS)])

        @pl.when(s == NS - 1)
        def _():
            pltpu.sync_copy(agg_sh.at[pl.ds(NS * RPS, RTAIL)],
                            agg_out.at[c].at[pl.ds(NS * RPS, RTAIL)])

        @pl.when(s == 0)
        def _():
            pltpu.sync_copy(cnt_sh, cnt_out.at[c].at[0])

    return k(x, src2d, dst2d, zrows, zcnt)


NB = 2000
GRID = N // NB


def _tc_body(p_ref, cnt_ref, xr_ref, wl_ref, wr_ref, w2_ref, bl_ref, b2_ref,
             o_ref, acc_ref):
    i = pl.program_id(0)
    ssum = p_ref[0] + p_ref[1]
    cnt = cnt_ref[0, 0, 0] + cnt_ref[1, 0, 0]
    cnt = jnp.maximum(cnt, 1.0)
    agg = ssum / cnt[:, None]
    dn = (((1,), (1,)), ((), ()))
    h = (lax.dot_general(agg, wl_ref[...], dn,
                         preferred_element_type=jnp.float32)
         + lax.dot_general(xr_ref[...], wr_ref[...], dn,
                           preferred_element_type=jnp.float32)
         + bl_ref[...])
    h = jnp.maximum(h, 0.0)
    hs = jnp.sum(h, axis=0, keepdims=True)

    @pl.when(i == 0)
    def _():
        acc_ref[...] = hs

    @pl.when(i > 0)
    def _():
        acc_ref[...] = acc_ref[...] + hs

    @pl.when(i == GRID - 1)
    def _():
        hmean = acc_ref[...] * (1.0 / N)
        o_ref[...] = (lax.dot_general(hmean, w2_ref[...], dn,
                                      preferred_element_type=jnp.float32)
                      + b2_ref[...])


def kernel(x, edge_index, W_l, b_l, W_r, W2, b2):
    src2d = edge_index[0].reshape(NW, SUPER, SUBN, C)
    dst2d = edge_index[1].reshape(NW, SUPER, SUBN, C)
    zrows = jnp.zeros((ZROWS, D), jnp.float32)
    zcnt = jnp.zeros((N,), jnp.float32)

    agg_p, cnt_p = _sc_segment_sum(x, src2d, dst2d, zrows, zcnt)

    y = pl.pallas_call(
        _tc_body,
        grid=(GRID,),
        in_specs=[
            pl.BlockSpec((NC, NB, D), lambda i: (0, i, 0)),
            pl.BlockSpec((NC, 1, 1, NB), lambda i: (0, i, 0, 0)),
            pl.BlockSpec((NB, D), lambda i: (i, 0)),
            pl.BlockSpec((D, D), lambda i: (0, 0)),
            pl.BlockSpec((D, D), lambda i: (0, 0)),
            pl.BlockSpec((D, D), lambda i: (0, 0)),
            pl.BlockSpec((1, D), lambda i: (0, 0)),
            pl.BlockSpec((1, D), lambda i: (0, 0)),
        ],
        out_specs=pl.BlockSpec((1, D), lambda i: (0, 0)),
        out_shape=jax.ShapeDtypeStruct((1, D), jnp.float32),
        scratch_shapes=[pltpu.VMEM((1, D), jnp.float32)],
    )(agg_p, cnt_p.reshape(NC, GRID, 1, NB), x, W_l, W_r, W2,
      b_l.reshape(1, D), b2.reshape(1, D))

    return y[0]


# speedup vs baseline: 1.1080x; 1.0467x over previous
import functools

import jax
import jax.numpy as jnp
from jax import lax
from jax.experimental import pallas as pl
from jax.experimental.pallas import tpu as pltpu
from jax.experimental.pallas import tpu_sc as plsc

N = 10000
E = 320000
D = 128

NC = 2
NS = 16
NW = NC * NS
EPW = E // NW
C = 80
NCHUNK = EPW // C
SUPER = 5
SUBN = NCHUNK // SUPER
NBUF = 3
RPS = 624
RTAIL = N - NS * RPS


def _sc_segment_sum(x, src2d, dst2d, zcnt):
    mesh = plsc.VectorSubcoreMesh(
        core_axis_name="c", subcore_axis_name="s",
        num_cores=NC, num_subcores=NS,
    )

    @functools.partial(
        pl.kernel,
        out_type=(
            jax.ShapeDtypeStruct((NC, N, D), jnp.float32),
            jax.ShapeDtypeStruct((NC, 1, N), jnp.float32),
        ),
        mesh=mesh,
        scratch_types=[
            pltpu.VMEM((2, SUBN, C), jnp.int32),
            pltpu.VMEM((2, SUBN, C), jnp.int32),
            pltpu.VMEM((NBUF, C, D), jnp.float32),
            pltpu.VMEM((128,), jnp.float32),
            pltpu.VMEM_SHARED((N, D), jnp.float32),
            pltpu.VMEM_SHARED((N,), jnp.float32),
            pltpu.SemaphoreType.DMA,
            pltpu.SemaphoreType.DMA,
            pltpu.SemaphoreType.DMA,
            pltpu.SemaphoreType.DMA,
        ],
    )
    def k(x_hbm, src_hbm, dst_hbm, zcnt_hbm,
          agg_out, cnt_out, src_v2, dst_v2, rows_v, ones_v,
          agg_sh, cnt_sh, gsem, ssem, osem, isem):
        c = lax.axis_index("c")
        s = lax.axis_index("s")
        wid = c * NS + s

        pltpu.async_copy(src_hbm.at[wid].at[0], src_v2.at[0], isem)
        pltpu.async_copy(dst_hbm.at[wid].at[0], dst_v2.at[0], isem)

        @pl.when(s == 0)
        def _():
            pltpu.sync_copy(zcnt_hbm, cnt_sh)

        for t in range(8):
            ones_v[pl.ds(t * 16, 16)] = jnp.ones((16,), jnp.float32)
        ones_c = ones_v.at[pl.ds(0, C)]

        pltpu.make_async_copy(src_hbm.at[wid].at[0], src_v2.at[0],
                              isem).wait()
        pltpu.make_async_copy(dst_hbm.at[wid].at[0], dst_v2.at[0],
                              isem).wait()
        for p in range(NBUF - 1):
            pltpu.async_copy(x_hbm.at[src_v2.at[0].at[p]], rows_v.at[p],
                             gsem)
        if SUPER > 1:
            pltpu.async_copy(src_hbm.at[wid].at[1], src_v2.at[1], isem)
            pltpu.async_copy(dst_hbm.at[wid].at[1], dst_v2.at[1], isem)

        zb = rows_v.at[NBUF - 1]

        def zbody(r, carry):
            for t in range(8):
                zb[r, pl.ds(t * 16, 16)] = jnp.zeros((16,), jnp.float32)
            return carry

        lax.fori_loop(0, C, zbody, 0, unroll=False)
        for kk in range(7):
            pltpu.async_copy(zb, agg_sh.at[pl.ds(s * RPS + kk * C, C)],
                             ssem)
        pltpu.async_copy(zb.at[pl.ds(0, RPS - 7 * C)],
                         agg_sh.at[pl.ds(s * RPS + 7 * C, RPS - 7 * C)],
                         ssem)

        @pl.when(s == NS - 1)
        def _():
            pltpu.async_copy(zb.at[pl.ds(0, RTAIL)],
                             agg_sh.at[pl.ds(NS * RPS, RTAIL)], ssem)

        for kk in range(7):
            pltpu.make_async_copy(zb, agg_sh.at[pl.ds(0, C)], ssem).wait()
        pltpu.make_async_copy(zb.at[pl.ds(0, RPS - 7 * C)],
                              agg_sh.at[pl.ds(0, RPS - 7 * C)],
                              ssem).wait()

        @pl.when(s == NS - 1)
        def _():
            pltpu.make_async_copy(zb.at[pl.ds(0, RTAIL)],
                                  agg_sh.at[pl.ds(0, RTAIL)], ssem).wait()

        plsc.subcore_barrier()

        for g in range(SUPER):
            gb = g % 2
            src_v = src_v2.at[gb]
            dst_v = dst_v2.at[gb]

            def body(j, carry):
                buf = lax.rem(j, NBUF)
                pltpu.make_async_copy(x_hbm.at[src_v.at[j]],
                                      rows_v.at[buf], gsem).wait()

                pltpu.async_copy(rows_v.at[buf], agg_sh.at[dst_v.at[j]],
                                 ssem, add=True)
                pltpu.async_copy(ones_c, cnt_sh.at[dst_v.at[j]],
                                 osem, add=True)

                nbuf = lax.rem(j + NBUF - 1, NBUF)

                @pl.when(j > 0)
                def _():
                    pltpu.make_async_copy(rows_v.at[nbuf],
                                          agg_sh.at[dst_v.at[j - 1]],
                                          ssem).wait()

                @pl.when(j < SUBN - (NBUF - 1))
                def _():
                    pltpu.async_copy(x_hbm.at[src_v.at[j + NBUF - 1]],
                                     rows_v.at[nbuf], gsem)

                return carry

            lax.fori_loop(0, SUBN, body, 0, unroll=False)

            pltpu.make_async_copy(rows_v.at[(SUBN - 1) % NBUF],
                                  agg_sh.at[dst_v.at[0]], ssem).wait()

            if g + 1 < SUPER:
                nsrc = src_v2.at[1 - gb]
                pltpu.make_async_copy(src_hbm.at[wid].at[g + 1], nsrc,
                                      isem).wait()
                pltpu.make_async_copy(dst_hbm.at[wid].at[g + 1],
                                      dst_v2.at[1 - gb], isem).wait()
                for p in range(NBUF - 1):
                    pltpu.async_copy(x_hbm.at[nsrc.at[p]], rows_v.at[p],
                                     gsem)

            def drain(j, carry):
                pltpu.make_async_copy(ones_c, cnt_sh.at[dst_v.at[0]],
                                      osem).wait()
                return carry

            lax.fori_loop(0, SUBN, drain, 0, unroll=False)

            if g + 2 < SUPER:
                pltpu.async_copy(src_hbm.at[wid].at[g + 2], src_v2.at[gb],
                                 isem)
                pltpu.async_copy(dst_hbm.at[wid].at[g + 2], dst_v2.at[gb],
                                 isem)

        plsc.subcore_barrier()

        pltpu.sync_copy(agg_sh.at[pl.ds(s * RPS, RPS)],
                        agg_out.at[c].at[pl.ds(s * RPS, RPS)])

        @pl.when(s == NS - 1)
        def _():
            pltpu.sync_copy(agg_sh.at[pl.ds(NS * RPS, RTAIL)],
                            agg_out.at[c].at[pl.ds(NS * RPS, RTAIL)])

        @pl.when(s == 0)
        def _():
            pltpu.sync_copy(cnt_sh, cnt_out.at[c].at[0])

    return k(x, src2d, dst2d, zcnt)


NB = 2000
GRID = N // NB


def _tc_body(p_ref, cnt_ref, xr_ref, wl_ref, wr_ref, w2_ref, bl_ref, b2_ref,
             o_ref, acc_ref):
    i = pl.program_id(0)
    ssum = p_ref[0] + p_ref[1]
    cnt = cnt_ref[0, 0, 0] + cnt_ref[1, 0, 0]
    cnt = jnp.maximum(cnt, 1.0)
    agg = ssum / cnt[:, None]
    dn = (((1,), (1,)), ((), ()))
    h = (lax.dot_general(agg, wl_ref[...], dn,
                         preferred_element_type=jnp.float32)
         + lax.dot_general(xr_ref[...], wr_ref[...], dn,
                           preferred_element_type=jnp.float32)
         + bl_ref[...])
    h = jnp.maximum(h, 0.0)
    hs = jnp.sum(h, axis=0, keepdims=True)

    @pl.when(i == 0)
    def _():
        acc_ref[...] = hs

    @pl.when(i > 0)
    def _():
        acc_ref[...] = acc_ref[...] + hs

    @pl.when(i == GRID - 1)
    def _():
        hmean = acc_ref[...] * (1.0 / N)
        o_ref[...] = (lax.dot_general(hmean, w2_ref[...], dn,
                                      preferred_element_type=jnp.float32)
                      + b2_ref[...])


def kernel(x, edge_index, W_l, b_l, W_r, W2, b2):
    src2d = edge_index[0].reshape(NW, SUPER, SUBN, C)
    dst2d = edge_index[1].reshape(NW, SUPER, SUBN, C)
    zcnt = jnp.zeros((N,), jnp.float32)

    agg_p, cnt_p = _sc_segment_sum(x, src2d, dst2d, zcnt)

    y = pl.pallas_call(
        _tc_body,
        grid=(GRID,),
        in_specs=[
            pl.BlockSpec((NC, NB, D), lambda i: (0, i, 0)),
            pl.BlockSpec((NC, 1, 1, NB), lambda i: (0, i, 0, 0)),
            pl.BlockSpec((NB, D), lambda i: (i, 0)),
            pl.BlockSpec((D, D), lambda i: (0, 0)),
            pl.BlockSpec((D, D), lambda i: (0, 0)),
            pl.BlockSpec((D, D), lambda i: (0, 0)),
            pl.BlockSpec((1, D), lambda i: (0, 0)),
            pl.BlockSpec((1, D), lambda i: (0, 0)),
        ],
        out_specs=pl.BlockSpec((1, D), lambda i: (0, 0)),
        out_shape=jax.ShapeDtypeStruct((1, D), jnp.float32),
        scratch_shapes=[pltpu.VMEM((1, D), jnp.float32)],
    )(agg_p, cnt_p.reshape(NC, GRID, 1, NB), x, W_l, W_r, W2,
      b_l.reshape(1, D), b2.reshape(1, D))

    return y[0]
